# Initial kernel scaffold; baseline (speedup 1.0000x reference)
import jax
import jax.numpy as jnp
from jax.experimental import pallas as pl


def kernel(img):
    raise NotImplementedError("write your pallas kernel here")


# speedup vs baseline: 2.4767x; 2.4767x over previous
import numpy as np
import jax
import jax.numpy as jnp
from jax.experimental import pallas as pl


def _disabled_tofs(tof_count, min_c, max_c, neighbor_p, seed=0):
    rng = np.random.default_rng(seed)
    count = int(rng.integers(min_c, max_c + 1))
    tof_list = rng.permutation(tof_count)
    first = int(rng.integers(1, tof_count))
    disabled = [first]
    tof_list = tof_list[tof_list != first]
    for _ in range(count - 1):
        r = float(rng.random())
        if r < neighbor_p:
            if r < neighbor_p / 2.0:
                offsets = (1, -1)
            else:
                offsets = (tof_count // 2, -(tof_count // 2))
            appended = False
            for d in list(disabled):
                for off in offsets:
                    cand = d + off
                    if cand in tof_list:
                        tof_list = tof_list[tof_list != cand]
                        disabled.append(int(cand))
                        appended = True
                        break
                if appended:
                    break
            if not appended:
                new = int(tof_list[0])
                tof_list = tof_list[tof_list != new]
                disabled.append(new)
        else:
            new = int(tof_list[0])
            tof_list = tof_list[tof_list != new]
            disabled.append(new)
    return sorted(int(x) for x in disabled)


_IDX = _disabled_tofs(2048, 1, 3, 0.5)

BLOCK_ROWS = 512


def _mask_copy_kernel(img_ref, out_ref):
    x = img_ref[...]
    cols = jax.lax.broadcasted_iota(jnp.int32, x.shape, dimension=1)
    keep = jnp.ones(x.shape, jnp.bool_)
    for c in _IDX:
        keep = keep & (cols != c)
    out_ref[...] = jnp.where(keep, x, jnp.float32(0.0))


def kernel(img):
    n_rows, n_cols = img.shape
    grid = (n_rows // BLOCK_ROWS,)
    return pl.pallas_call(
        _mask_copy_kernel,
        grid=grid,
        in_specs=[pl.BlockSpec((BLOCK_ROWS, n_cols), lambda i: (i, 0))],
        out_specs=pl.BlockSpec((BLOCK_ROWS, n_cols), lambda i: (i, 0)),
        out_shape=jax.ShapeDtypeStruct((n_rows, n_cols), img.dtype),
    )(img)


# speedup vs baseline: 2.5214x; 1.0181x over previous
import numpy as np
import jax
import jax.numpy as jnp
from jax.experimental import pallas as pl


def _disabled_tofs(tof_count, min_c, max_c, neighbor_p, seed=0):
    rng = np.random.default_rng(seed)
    count = int(rng.integers(min_c, max_c + 1))
    tof_list = rng.permutation(tof_count)
    first = int(rng.integers(1, tof_count))
    disabled = [first]
    tof_list = tof_list[tof_list != first]
    for _ in range(count - 1):
        r = float(rng.random())
        if r < neighbor_p:
            if r < neighbor_p / 2.0:
                offsets = (1, -1)
            else:
                offsets = (tof_count // 2, -(tof_count // 2))
            appended = False
            for d in list(disabled):
                for off in offsets:
                    cand = d + off
                    if cand in tof_list:
                        tof_list = tof_list[tof_list != cand]
                        disabled.append(int(cand))
                        appended = True
                        break
                if appended:
                    break
            if not appended:
                new = int(tof_list[0])
                tof_list = tof_list[tof_list != new]
                disabled.append(new)
        else:
            new = int(tof_list[0])
            tof_list = tof_list[tof_list != new]
            disabled.append(new)
    return sorted(int(x) for x in disabled)


_IDX = _disabled_tofs(2048, 1, 3, 0.5)

BLOCK_ROWS = 1024


def _mask_copy_kernel(img_ref, out_ref):
    x = img_ref[...]
    cols = jax.lax.broadcasted_iota(jnp.int32, x.shape, dimension=1)
    keep = jnp.ones(x.shape, jnp.bool_)
    for c in _IDX:
        keep = keep & (cols != c)
    out_ref[...] = jnp.where(keep, x, jnp.float32(0.0))


def kernel(img):
    n_rows, n_cols = img.shape
    grid = (n_rows // BLOCK_ROWS,)
    return pl.pallas_call(
        _mask_copy_kernel,
        grid=grid,
        in_specs=[pl.BlockSpec((BLOCK_ROWS, n_cols), lambda i: (i, 0))],
        out_specs=pl.BlockSpec((BLOCK_ROWS, n_cols), lambda i: (i, 0)),
        out_shape=jax.ShapeDtypeStruct((n_rows, n_cols), img.dtype),
    )(img)
